# baseline (speedup 1.0000x reference)
import jax
import jax.numpy as jnp
from jax.experimental import pallas as pl
from jax.experimental.pallas import tpu as pltpu

LATENT = 64
NE = 10
TILE = 16384


def _dott(wt, x):
    return jax.lax.dot_general(wt, x, (((0,), (0,)), ((), ())))


def _body(xt_ref, w1t_ref, b1_ref, wmt_ref, bm_ref, wlt_ref, bl_ref,
          cb_ref, cbn_ref, r1t_ref, rb1_ref, r2t_ref, rb2_ref,
          d1t_ref, db1_ref, d2t_ref, db2_ref,
          dect_ref, meant_ref, logvart_ref, quantt_ref, idx_ref):
    x = xt_ref[...]
    h = jnp.maximum(_dott(w1t_ref[...], x) + b1_ref[...], 0.0)
    mean = _dott(wmt_ref[...], h) + bm_ref[...]
    logvar = _dott(wlt_ref[...], h) + bl_ref[...]
    meant_ref[...] = mean
    logvart_ref[...] = logvar
    d = cbn_ref[...] - 2.0 * jnp.dot(cb_ref[...], mean)
    dmin = jnp.min(d, axis=0, keepdims=True)
    iota = jax.lax.broadcasted_iota(jnp.int32, d.shape, 0)
    idx = jnp.min(jnp.where(d == dmin, iota, NE), axis=0)
    idx_ref[...] = idx
    onehot = (iota == idx[None, :]).astype(jnp.float32)
    quant = _dott(cb_ref[...], onehot)
    quantt_ref[...] = quant
    r = jnp.maximum(_dott(r1t_ref[...], quant) + rb1_ref[...], 0.0)
    r = _dott(r2t_ref[...], r) + rb2_ref[...]
    qr = r + quant
    dd = jnp.maximum(_dott(d1t_ref[...], qr) + db1_ref[...], 0.0)
    dect_ref[...] = jax.nn.sigmoid(_dott(d2t_ref[...], dd) + db2_ref[...])


def kernel(inputs, enc_fc1_w, enc_fc1_b, enc_mean_w, enc_mean_b,
           enc_logvar_w, enc_logvar_b, codebook,
           res_fc1_w, res_fc1_b, res_fc2_w, res_fc2_b,
           dec_fc1_w, dec_fc1_b, dec_fc2_w, dec_fc2_b):
    B, IN = inputs.shape
    tile = TILE if B % TILE == 0 else B
    grid = (B // tile,)

    col = lambda b: b.reshape(-1, 1)
    args = (
        inputs.T,
        enc_fc1_w.T, col(enc_fc1_b),
        enc_mean_w.T, col(enc_mean_b),
        enc_logvar_w.T, col(enc_logvar_b),
        codebook,
        col(jnp.sum(codebook * codebook, axis=1)),
        res_fc1_w.T, col(res_fc1_b),
        res_fc2_w.T, col(res_fc2_b),
        dec_fc1_w.T, col(dec_fc1_b),
        dec_fc2_w.T, col(dec_fc2_b),
    )

    x_spec = pl.BlockSpec((IN, tile), lambda i: (0, i))
    full = lambda a: pl.BlockSpec(a.shape, lambda i: (0,) * a.ndim)
    in_specs = [x_spec] + [full(a) for a in args[1:]]
    out_specs = [
        pl.BlockSpec((IN, tile), lambda i: (0, i)),
        pl.BlockSpec((LATENT, tile), lambda i: (0, i)),
        pl.BlockSpec((LATENT, tile), lambda i: (0, i)),
        pl.BlockSpec((LATENT, tile), lambda i: (0, i)),
        pl.BlockSpec((tile,), lambda i: (i,)),
    ]
    out_shape = [
        jax.ShapeDtypeStruct((IN, B), jnp.float32),
        jax.ShapeDtypeStruct((LATENT, B), jnp.float32),
        jax.ShapeDtypeStruct((LATENT, B), jnp.float32),
        jax.ShapeDtypeStruct((LATENT, B), jnp.float32),
        jax.ShapeDtypeStruct((B,), jnp.int32),
    ]
    dect, meant, logvart, quantt, idx = pl.pallas_call(
        _body,
        grid=grid,
        in_specs=in_specs,
        out_specs=out_specs,
        out_shape=out_shape,
        compiler_params=pltpu.CompilerParams(
            dimension_semantics=("parallel",)),
    )(*args)
    return (dect.T, meant.T, logvart.T, quantt.T, idx)


# speedup vs baseline: 1.0050x; 1.0050x over previous
import jax
import jax.numpy as jnp
from jax.experimental import pallas as pl
from jax.experimental.pallas import tpu as pltpu

LATENT = 64
NE = 10
TILE = 16384


def _body(xt_ref, w1_ref, b1_ref, wm_ref, bm_ref, wl_ref, bl_ref,
          cb_ref, cbt_ref, cbn_ref, r1_ref, rb1_ref, r2_ref, rb2_ref,
          d1_ref, db1_ref, d2_ref, db2_ref,
          dect_ref, meant_ref, logvart_ref, quantt_ref, idx_ref):
    x = xt_ref[...]
    h = jnp.maximum(jnp.dot(w1_ref[...], x) + b1_ref[...], 0.0)
    mean = jnp.dot(wm_ref[...], h) + bm_ref[...]
    logvar = jnp.dot(wl_ref[...], h) + bl_ref[...]
    meant_ref[...] = mean
    logvart_ref[...] = logvar
    d = cbn_ref[...] - 2.0 * jnp.dot(cb_ref[...], mean)
    dmin = jnp.min(d, axis=0, keepdims=True)
    iota = jax.lax.broadcasted_iota(jnp.int32, d.shape, 0)
    idx = jnp.min(jnp.where(d == dmin, iota, NE), axis=0)
    idx_ref[...] = idx
    onehot = (iota == idx[None, :]).astype(jnp.float32)
    quant = jnp.dot(cbt_ref[...], onehot)
    quantt_ref[...] = quant
    r = jnp.maximum(jnp.dot(r1_ref[...], quant) + rb1_ref[...], 0.0)
    r = jnp.dot(r2_ref[...], r) + rb2_ref[...]
    qr = r + quant
    dd = jnp.maximum(jnp.dot(d1_ref[...], qr) + db1_ref[...], 0.0)
    dect_ref[...] = jax.nn.sigmoid(jnp.dot(d2_ref[...], dd) + db2_ref[...])


def kernel(inputs, enc_fc1_w, enc_fc1_b, enc_mean_w, enc_mean_b,
           enc_logvar_w, enc_logvar_b, codebook,
           res_fc1_w, res_fc1_b, res_fc2_w, res_fc2_b,
           dec_fc1_w, dec_fc1_b, dec_fc2_w, dec_fc2_b):
    B, IN = inputs.shape
    tile = TILE if B % TILE == 0 else B
    grid = (B // tile,)

    col = lambda b: b.reshape(-1, 1)
    args = (
        inputs.T,
        enc_fc1_w, col(enc_fc1_b),
        enc_mean_w, col(enc_mean_b),
        enc_logvar_w, col(enc_logvar_b),
        codebook, codebook.T,
        col(jnp.sum(codebook * codebook, axis=1)),
        res_fc1_w, col(res_fc1_b),
        res_fc2_w, col(res_fc2_b),
        dec_fc1_w, col(dec_fc1_b),
        dec_fc2_w, col(dec_fc2_b),
    )

    x_spec = pl.BlockSpec((IN, tile), lambda i: (0, i))
    full = lambda a: pl.BlockSpec(a.shape, lambda i: (0,) * a.ndim)
    in_specs = [x_spec] + [full(a) for a in args[1:]]
    out_specs = [
        pl.BlockSpec((IN, tile), lambda i: (0, i)),
        pl.BlockSpec((LATENT, tile), lambda i: (0, i)),
        pl.BlockSpec((LATENT, tile), lambda i: (0, i)),
        pl.BlockSpec((LATENT, tile), lambda i: (0, i)),
        pl.BlockSpec((tile,), lambda i: (i,)),
    ]
    out_shape = [
        jax.ShapeDtypeStruct((IN, B), jnp.float32),
        jax.ShapeDtypeStruct((LATENT, B), jnp.float32),
        jax.ShapeDtypeStruct((LATENT, B), jnp.float32),
        jax.ShapeDtypeStruct((LATENT, B), jnp.float32),
        jax.ShapeDtypeStruct((B,), jnp.int32),
    ]
    dect, meant, logvart, quantt, idx = pl.pallas_call(
        _body,
        grid=grid,
        in_specs=in_specs,
        out_specs=out_specs,
        out_shape=out_shape,
        compiler_params=pltpu.CompilerParams(
            dimension_semantics=("parallel",)),
    )(*args)
    return (dect.T, meant.T, logvart.T, quantt.T, idx)
